# Initial kernel scaffold; baseline (speedup 1.0000x reference)
import functools

import jax
import jax.numpy as jnp
from jax import lax
from jax.experimental import pallas as pl
from jax.experimental.pallas import tpu as pltpu
from jax.experimental.pallas import tpu_sc as plsc

N_NODES = 10000
N_EDGES = 320000
D_IN = 128
H2 = 256
H = 128

NC = 2
NS = 16
NW = NC * NS
EPW = N_EDGES // NW
K = 125
NB = EPW // K
RPS = N_NODES // NS
CPAD = 10240
CPS = CPAD // NS

_HIGH = jax.lax.Precision.HIGHEST


def _seg_sum_sc(feats, src3, dst3, z2d, z1d, ones, with_counts):
    D = feats.shape[1]
    mesh = plsc.VectorSubcoreMesh(core_axis_name="c", subcore_axis_name="s")
    out_type = [jax.ShapeDtypeStruct((NC, N_NODES, D), jnp.float32)]
    scratch = [
        pltpu.VMEM((NB, K), jnp.int32),
        pltpu.VMEM((NB, K), jnp.int32),
        pltpu.VMEM((K, D), jnp.float32),
        pltpu.VMEM_SHARED((N_NODES, D), jnp.float32),
    ]
    if with_counts:
        out_type.append(jax.ShapeDtypeStruct((NC, CPAD), jnp.float32))
        scratch += [
            pltpu.VMEM((K,), jnp.float32),
            pltpu.VMEM_SHARED((CPAD,), jnp.float32),
        ]

    def body(x_hbm, src_hbm, dst_hbm, z2_hbm, z1_hbm, ones_hbm, *rest):
        if with_counts:
            agg_hbm, cnt_hbm, src_v, dst_v, rows_v, acc_sp, ones_v, cnt_sp = rest
        else:
            agg_hbm, src_v, dst_v, rows_v, acc_sp = rest
        c = lax.axis_index("c")
        s = lax.axis_index("s")
        w = s * NC + c
        pltpu.sync_copy(z2_hbm, acc_sp.at[pl.ds(s * RPS, RPS)])
        pltpu.sync_copy(src_hbm.at[w], src_v)
        pltpu.sync_copy(dst_hbm.at[w], dst_v)
        if with_counts:
            pltpu.sync_copy(z1_hbm, cnt_sp.at[pl.ds(s * CPS, CPS)])
            pltpu.sync_copy(ones_hbm, ones_v)
        plsc.subcore_barrier()

        @pl.loop(0, NB)
        def _(j):
            pltpu.sync_copy(x_hbm.at[src_v.at[j]], rows_v)
            pltpu.sync_copy(rows_v, acc_sp.at[dst_v.at[j]], add=True)
            if with_counts:
                pltpu.sync_copy(ones_v, cnt_sp.at[dst_v.at[j]], add=True)

        plsc.subcore_barrier()
        pltpu.sync_copy(acc_sp.at[pl.ds(s * RPS, RPS)],
                        agg_hbm.at[c, pl.ds(s * RPS, RPS)])
        if with_counts:
            pltpu.sync_copy(cnt_sp.at[pl.ds(s * CPS, CPS)],
                            cnt_hbm.at[c, pl.ds(s * CPS, CPS)])

    return pl.kernel(body, out_type=tuple(out_type), mesh=mesh,
                     scratch_types=scratch)(feats, src3, dst3, z2d, z1d, ones)


R = 2000
GRID = N_NODES // R


def _tc1_body(a_ref, cnt_ref, x_ref, w1l_ref, b1_ref, w1r_ref, w2l_ref,
              w2r_ref, b2_ref, g_ref, r_ref):
    a = a_ref[0] + a_ref[1]
    cnt = cnt_ref[:, 0:1] + cnt_ref[:, 1:2]
    inv = 1.0 / jnp.maximum(cnt, 1.0)
    mean = a * inv
    t = (jnp.dot(mean, w1l_ref[...], precision=_HIGH,
                 preferred_element_type=jnp.float32)
         + jnp.dot(x_ref[...], w1r_ref[...], precision=_HIGH,
                   preferred_element_type=jnp.float32)
         + b1_ref[...])
    h = jnp.maximum(t, 0.0)
    g_ref[...] = jnp.dot(h, w2l_ref[...], precision=_HIGH,
                         preferred_element_type=jnp.float32)
    r_ref[...] = jnp.dot(h, w2r_ref[...], precision=_HIGH,
                         preferred_element_type=jnp.float32) + b2_ref[...]


def _tc1(agg1, cnt2, x, w1l_t, b1, w1r_t, w2l_t, w2r_t, b2):
    return pl.pallas_call(
        _tc1_body,
        grid=(GRID,),
        in_specs=[
            pl.BlockSpec((NC, R, D_IN), lambda i: (0, i, 0)),
            pl.BlockSpec((R, NC), lambda i: (i, 0)),
            pl.BlockSpec((R, D_IN), lambda i: (i, 0)),
            pl.BlockSpec((D_IN, H2), lambda i: (0, 0)),
            pl.BlockSpec((1, H2), lambda i: (0, 0)),
            pl.BlockSpec((D_IN, H2), lambda i: (0, 0)),
            pl.BlockSpec((H2, H), lambda i: (0, 0)),
            pl.BlockSpec((H2, H), lambda i: (0, 0)),
            pl.BlockSpec((1, H), lambda i: (0, 0)),
        ],
        out_specs=[
            pl.BlockSpec((R, H), lambda i: (i, 0)),
            pl.BlockSpec((R, H), lambda i: (i, 0)),
        ],
        out_shape=[
            jax.ShapeDtypeStruct((N_NODES, H), jnp.float32),
            jax.ShapeDtypeStruct((N_NODES, H), jnp.float32),
        ],
    )(agg1, cnt2, x, w1l_t, b1, w1r_t, w2l_t, w2r_t, b2)


def _tc2_body(a_ref, cnt_ref, r_ref, o_ref):
    a = a_ref[0] + a_ref[1]
    cnt = cnt_ref[:, 0:1] + cnt_ref[:, 1:2]
    inv = 1.0 / jnp.maximum(cnt, 1.0)
    t = a * inv + r_ref[...]
    m = jnp.max(t, axis=1, keepdims=True)
    e = jnp.exp(t - m)
    lse = jnp.log(jnp.sum(e, axis=1, keepdims=True))
    o_ref[...] = t - m - lse


def _tc2(agg2, cnt2, r):
    return pl.pallas_call(
        _tc2_body,
        grid=(GRID,),
        in_specs=[
            pl.BlockSpec((NC, R, H), lambda i: (0, i, 0)),
            pl.BlockSpec((R, NC), lambda i: (i, 0)),
            pl.BlockSpec((R, H), lambda i: (i, 0)),
        ],
        out_specs=pl.BlockSpec((R, H), lambda i: (i, 0)),
        out_shape=jax.ShapeDtypeStruct((N_NODES, H), jnp.float32),
    )(agg2, cnt2, r)


def kernel(x, edge_index, W1_l, b1_l, W1_r, W2_l, b2_l, W2_r):
    src3 = edge_index[0].astype(jnp.int32).reshape(NW, NB, K)
    dst3 = edge_index[1].astype(jnp.int32).reshape(NW, NB, K)
    z2d = jnp.zeros((RPS, D_IN), jnp.float32)
    z1d = jnp.zeros((CPS,), jnp.float32)
    ones = jnp.ones((K,), jnp.float32)

    agg1, cnt = _seg_sum_sc(x, src3, dst3, z2d, z1d, ones, with_counts=True)
    cnt2 = cnt[:, :N_NODES].T

    g, r = _tc1(agg1, cnt2, x,
                W1_l.T, b1_l.reshape(1, H2), W1_r.T,
                W2_l.T, W2_r.T, b2_l.reshape(1, H))

    (agg2,) = _seg_sum_sc(g, src3, dst3, z2d, z1d, ones, with_counts=False)
    return _tc2(agg2, cnt2, r)


# speedup vs baseline: 9.3631x; 9.3631x over previous
import functools

import jax
import jax.numpy as jnp
from jax import lax
from jax.experimental import pallas as pl
from jax.experimental.pallas import tpu as pltpu
from jax.experimental.pallas import tpu_sc as plsc

N_NODES = 10000
N_EDGES = 320000
D_IN = 128
H2 = 256
H = 128

NC = 2
NS = 16
NW = NC * NS
EPW = N_EDGES // NW
K = 125
NB = EPW // K
NPAD = 10240
RPS = NPAD // NS
CPAD = NPAD
CPS = CPAD // NS

_HIGH = jax.lax.Precision.HIGHEST


def _seg_sum_sc(feats, src3, dst3, z2d, z1d, ones, with_counts):
    D = feats.shape[1]
    mesh = plsc.VectorSubcoreMesh(core_axis_name="c", subcore_axis_name="s")
    out_type = [jax.ShapeDtypeStruct((NC, NPAD, D), jnp.float32)]
    scratch = [
        pltpu.VMEM((NB, K), jnp.int32),
        pltpu.VMEM((NB, K), jnp.int32),
        pltpu.VMEM((K, D), jnp.float32),
        pltpu.VMEM_SHARED((NPAD, D), jnp.float32),
    ]
    if with_counts:
        out_type.append(jax.ShapeDtypeStruct((NC, CPAD), jnp.float32))
        scratch += [
            pltpu.VMEM((K,), jnp.float32),
            pltpu.VMEM_SHARED((CPAD,), jnp.float32),
        ]

    def body(x_hbm, src_hbm, dst_hbm, z2_hbm, z1_hbm, ones_hbm, *rest):
        if with_counts:
            agg_hbm, cnt_hbm, src_v, dst_v, rows_v, acc_sp, ones_v, cnt_sp = rest
        else:
            agg_hbm, src_v, dst_v, rows_v, acc_sp = rest
        c = lax.axis_index("c")
        s = lax.axis_index("s")
        w = s * NC + c
        pltpu.sync_copy(z2_hbm, acc_sp.at[pl.ds(s * RPS, RPS)])
        pltpu.sync_copy(src_hbm.at[w], src_v)
        pltpu.sync_copy(dst_hbm.at[w], dst_v)
        if with_counts:
            pltpu.sync_copy(z1_hbm, cnt_sp.at[pl.ds(s * CPS, CPS)])
            pltpu.sync_copy(ones_hbm, ones_v)
        plsc.subcore_barrier()

        @pl.loop(0, NB)
        def _(j):
            pltpu.sync_copy(x_hbm.at[src_v.at[j]], rows_v)
            pltpu.sync_copy(rows_v, acc_sp.at[dst_v.at[j]], add=True)
            if with_counts:
                pltpu.sync_copy(ones_v, cnt_sp.at[dst_v.at[j]], add=True)

        plsc.subcore_barrier()
        pltpu.sync_copy(acc_sp.at[pl.ds(s * RPS, RPS)],
                        agg_hbm.at[c, pl.ds(s * RPS, RPS)])
        if with_counts:
            pltpu.sync_copy(cnt_sp.at[pl.ds(s * CPS, CPS)],
                            cnt_hbm.at[c, pl.ds(s * CPS, CPS)])

    return pl.kernel(body, out_type=tuple(out_type), mesh=mesh,
                     scratch_types=scratch)(feats, src3, dst3, z2d, z1d, ones)


R = 2000
GRID = N_NODES // R


def _tc1_body(a_ref, cnt_ref, x_ref, w1l_ref, b1_ref, w1r_ref, w2l_ref,
              w2r_ref, b2_ref, g_ref, r_ref):
    a = a_ref[0] + a_ref[1]
    cnt = cnt_ref[:, 0:1] + cnt_ref[:, 1:2]
    inv = 1.0 / jnp.maximum(cnt, 1.0)
    mean = a * inv
    t = (jnp.dot(mean, w1l_ref[...], precision=_HIGH,
                 preferred_element_type=jnp.float32)
         + jnp.dot(x_ref[...], w1r_ref[...], precision=_HIGH,
                   preferred_element_type=jnp.float32)
         + b1_ref[...])
    h = jnp.maximum(t, 0.0)
    g_ref[...] = jnp.dot(h, w2l_ref[...], precision=_HIGH,
                         preferred_element_type=jnp.float32)
    r_ref[...] = jnp.dot(h, w2r_ref[...], precision=_HIGH,
                         preferred_element_type=jnp.float32) + b2_ref[...]


def _tc1(agg1, cnt2, x, w1l_t, b1, w1r_t, w2l_t, w2r_t, b2):
    return pl.pallas_call(
        _tc1_body,
        grid=(GRID,),
        in_specs=[
            pl.BlockSpec((NC, R, D_IN), lambda i: (0, i, 0)),
            pl.BlockSpec((R, NC), lambda i: (i, 0)),
            pl.BlockSpec((R, D_IN), lambda i: (i, 0)),
            pl.BlockSpec((D_IN, H2), lambda i: (0, 0)),
            pl.BlockSpec((1, H2), lambda i: (0, 0)),
            pl.BlockSpec((D_IN, H2), lambda i: (0, 0)),
            pl.BlockSpec((H2, H), lambda i: (0, 0)),
            pl.BlockSpec((H2, H), lambda i: (0, 0)),
            pl.BlockSpec((1, H), lambda i: (0, 0)),
        ],
        out_specs=[
            pl.BlockSpec((R, H), lambda i: (i, 0)),
            pl.BlockSpec((R, H), lambda i: (i, 0)),
        ],
        out_shape=[
            jax.ShapeDtypeStruct((N_NODES, H), jnp.float32),
            jax.ShapeDtypeStruct((N_NODES, H), jnp.float32),
        ],
    )(agg1, cnt2, x, w1l_t, b1, w1r_t, w2l_t, w2r_t, b2)


def _tc2_body(a_ref, cnt_ref, r_ref, o_ref):
    a = a_ref[0] + a_ref[1]
    cnt = cnt_ref[:, 0:1] + cnt_ref[:, 1:2]
    inv = 1.0 / jnp.maximum(cnt, 1.0)
    t = a * inv + r_ref[...]
    m = jnp.max(t, axis=1, keepdims=True)
    e = jnp.exp(t - m)
    lse = jnp.log(jnp.sum(e, axis=1, keepdims=True))
    o_ref[...] = t - m - lse


def _tc2(agg2, cnt2, r):
    return pl.pallas_call(
        _tc2_body,
        grid=(GRID,),
        in_specs=[
            pl.BlockSpec((NC, R, H), lambda i: (0, i, 0)),
            pl.BlockSpec((R, NC), lambda i: (i, 0)),
            pl.BlockSpec((R, H), lambda i: (i, 0)),
        ],
        out_specs=pl.BlockSpec((R, H), lambda i: (i, 0)),
        out_shape=jax.ShapeDtypeStruct((N_NODES, H), jnp.float32),
    )(agg2, cnt2, r)


def kernel(x, edge_index, W1_l, b1_l, W1_r, W2_l, b2_l, W2_r):
    src3 = edge_index[0].astype(jnp.int32).reshape(NW, NB, K)
    dst3 = edge_index[1].astype(jnp.int32).reshape(NW, NB, K)
    z2d = jnp.zeros((RPS, D_IN), jnp.float32)
    z1d = jnp.zeros((CPS,), jnp.float32)
    ones = jnp.ones((K,), jnp.float32)

    agg1, cnt = _seg_sum_sc(x, src3, dst3, z2d, z1d, ones, with_counts=True)
    cnt2 = cnt[:, :N_NODES].T

    g, r = _tc1(agg1, cnt2, x,
                W1_l.T, b1_l.reshape(1, H2), W1_r.T,
                W2_l.T, W2_r.T, b2_l.reshape(1, H))

    (agg2,) = _seg_sum_sc(g, src3, dst3, z2d, z1d, ones, with_counts=False)
    return _tc2(agg2, cnt2, r)
